# baseline (speedup 1.0000x reference)
import functools

import jax
import jax.numpy as jnp
from jax import lax
from jax.experimental import pallas as pl
from jax.experimental.pallas import tpu as pltpu
from jax.experimental.pallas import tpu_sc as plsc

_CTR = (((1,), (1,)), ((), ()))


def _mlp_table_kernel(x_ref, w1_ref, b1_ref, w2_ref, b2_ref, out_ref):
    xb = x_ref[...]
    h = lax.dot_general(xb, w1_ref[0], _CTR, preferred_element_type=jnp.float32)
    h = jnp.maximum(h + b1_ref[0], 0.0)
    y = lax.dot_general(h, w2_ref[0], _CTR, preferred_element_type=jnp.float32)
    out_ref[...] = y + b2_ref[0]


def _mlp_table(x, W1a, b1a, W2a, b2a, bn):
    n, d = x.shape
    nk = W1a.shape[0]
    grid = (nk, n // bn)
    return pl.pallas_call(
        _mlp_table_kernel,
        grid=grid,
        in_specs=[
            pl.BlockSpec((bn, d), lambda k, i: (i, 0)),
            pl.BlockSpec((1, d, d), lambda k, i: (k, 0, 0)),
            pl.BlockSpec((1, 1, d), lambda k, i: (k, 0, 0)),
            pl.BlockSpec((1, d, d), lambda k, i: (k, 0, 0)),
            pl.BlockSpec((1, 1, d), lambda k, i: (k, 0, 0)),
        ],
        out_specs=pl.BlockSpec((bn, d), lambda k, i, _nb=n // bn: (k * _nb + i, 0)),
        out_shape=jax.ShapeDtypeStruct((nk * n, d), jnp.float32),
    )(x, W1a, b1a.reshape(nk, 1, d), W2a, b2a.reshape(nk, 1, d))


def _edge_idx_kernel(n_nodes, ei_ref, et_ref, gidx_ref, sidx_ref):
    src = ei_ref[0, :]
    dst = ei_ref[1, :]
    et = et_ref[...]
    gidx_ref[0, :] = et * n_nodes + src
    gidx_ref[1, :] = 2 * n_nodes + et * n_nodes + dst
    sidx_ref[0, :] = dst
    sidx_ref[1, :] = src


def _edge_indices(edge_index, edge_type, n_nodes):
    e = edge_type.shape[0]
    return pl.pallas_call(
        functools.partial(_edge_idx_kernel, n_nodes),
        out_shape=(
            jax.ShapeDtypeStruct((2, e), jnp.int32),
            jax.ShapeDtypeStruct((2, e), jnp.int32),
        ),
    )(edge_index, edge_type)


def _make_sc_agg(n_pad, d, chunks, csz, n_sub):
    rpt = n_pad // n_sub
    mesh = plsc.VectorSubcoreMesh(core_axis_name="c", subcore_axis_name="s")

    @functools.partial(
        pl.kernel,
        mesh=mesh,
        out_type=jax.ShapeDtypeStruct((2, n_pad, d), jnp.float32),
        scratch_types=[
            pltpu.VMEM((2, chunks // 2, csz), jnp.int32),
            pltpu.VMEM((csz, d), jnp.float32),
            pltpu.VMEM((csz, d), jnp.float32),
            pltpu.SemaphoreType.DMA,
            pltpu.SemaphoreType.DMA,
            pltpu.SemaphoreType.DMA,
            pltpu.SemaphoreType.DMA,
            pltpu.VMEM_SHARED((n_pad, d), jnp.float32),
        ],
    )
    def sc_agg(yy_hbm, kidx_hbm, out_hbm,
               idx_v, rows0, rows1, semg0, semg1, sems0, sems1, acc):
        c = lax.axis_index("c")
        s = lax.axis_index("s")
        zstrip = min(csz - csz % 8, rpt)
        zv = jnp.zeros((16,), jnp.float32)

        def zrow(r, carry):
            def zcol(q, carry2):
                rows0[r, pl.ds(q * 16, 16)] = zv
                return carry2
            return lax.fori_loop(0, d // 16, zcol, carry)

        lax.fori_loop(0, zstrip, zrow, 0)
        nfull = rpt // zstrip
        tail = rpt - nfull * zstrip

        def zdma(k, carry):
            pltpu.sync_copy(rows0.at[pl.ds(0, zstrip)],
                            acc.at[pl.ds(s * rpt + k * zstrip, zstrip)])
            return carry

        lax.fori_loop(0, nfull, zdma, 0)
        if tail:
            pltpu.sync_copy(rows0.at[pl.ds(0, tail)],
                            acc.at[pl.ds(s * rpt + nfull * zstrip, tail)])
        plsc.subcore_barrier()

        pchunks = chunks // 2

        def phase(p, carry):
            pltpu.sync_copy(kidx_hbm.at[c, s, p], idx_v)
            pltpu.async_copy(yy_hbm.at[idx_v.at[0, 0]], rows0, semg0)
            pltpu.async_copy(yy_hbm.at[idx_v.at[0, 1]], rows1, semg1)

            def body(j, carry2):
                i = 2 * j
                pltpu.make_async_copy(yy_hbm.at[idx_v.at[0, i]], rows0,
                                      semg0).wait()
                pltpu.async_copy(rows0, acc.at[idx_v.at[1, i]], sems0,
                                 add=True)
                pltpu.make_async_copy(yy_hbm.at[idx_v.at[0, i + 1]], rows1,
                                      semg1).wait()
                pltpu.async_copy(rows1, acc.at[idx_v.at[1, i + 1]], sems1,
                                 add=True)
                pltpu.make_async_copy(rows0, acc.at[idx_v.at[1, i]],
                                      sems0).wait()

                @pl.when(i + 2 < pchunks)
                def _():
                    pltpu.async_copy(yy_hbm.at[idx_v.at[0, i + 2]], rows0,
                                     semg0)

                pltpu.make_async_copy(rows1, acc.at[idx_v.at[1, i + 1]],
                                      sems1).wait()

                @pl.when(i + 3 < pchunks)
                def _():
                    pltpu.async_copy(yy_hbm.at[idx_v.at[0, i + 3]], rows1,
                                     semg1)

                return carry2

            lax.fori_loop(0, pchunks // 2, body, 0)
            return carry

        lax.fori_loop(0, 2, phase, 0)
        plsc.subcore_barrier()
        pltpu.sync_copy(acc.at[pl.ds(s * rpt, rpt)],
                        out_hbm.at[c, pl.ds(s * rpt, rpt)])

    return sc_agg


def _finish_kernel(x_ref, aggs_ref, aggg_ref, ws_ref, bs_ref, wg_ref, bg_ref,
                   pw_ref, pb_ref, ow_ref, ob_ref, out_ref):
    xb = x_ref[...]
    hu = jnp.tanh(lax.dot_general(xb, ws_ref[...], _CTR,
                                  preferred_element_type=jnp.float32)
                  + bs_ref[...] + aggs_ref[...])
    hv = jnp.tanh(lax.dot_general(xb, wg_ref[...], _CTR,
                                  preferred_element_type=jnp.float32)
                  + bg_ref[...] + aggg_ref[...])
    gate = jax.nn.sigmoid(
        jnp.dot(hu, pw_ref[...], preferred_element_type=jnp.float32)
        + pb_ref[0, 0])
    h = gate * hu + (1.0 - gate) * hv
    out_ref[...] = jax.nn.sigmoid(
        jnp.dot(h, ow_ref[...], preferred_element_type=jnp.float32)
        + ob_ref[0, 0])


def _finish(x, agg_s, agg_g, Ws, bs, Wg, bg, pW, pb, oW, ob, bn):
    n, d = x.shape
    full = pl.BlockSpec((d, d), lambda i: (0, 0))
    row = pl.BlockSpec((1, d), lambda i: (0, 0))
    col = pl.BlockSpec((d, 1), lambda i: (0, 0))
    scal = pl.BlockSpec((1, 1), lambda i: (0, 0))
    blk = pl.BlockSpec((bn, d), lambda i: (i, 0))
    return pl.pallas_call(
        _finish_kernel,
        grid=(n // bn,),
        in_specs=[blk, blk, blk, full, row, full, row, col, scal, col, scal],
        out_specs=pl.BlockSpec((bn, 1), lambda i: (i, 0)),
        out_shape=jax.ShapeDtypeStruct((n, 1), jnp.float32),
    )(x, agg_s, agg_g, Ws, bs.reshape(1, d), Wg, bg.reshape(1, d),
      pW.reshape(d, 1), pb.reshape(1, 1), oW.reshape(d, 1), ob.reshape(1, 1))


def kernel(x, edge_index, edge_type, W1_s, b1_s, W2_s, b2_s,
           W1_g, b1_g, W2_g, b2_g, Ws, bs, Wg, bg, pW, pb, oW, ob):
    n, d = x.shape
    e = edge_type.shape[0]
    n_sub = 16
    ept = e // n_sub
    csz = 125
    chunks = ept // csz

    W1a = jnp.concatenate([W1_s, W1_g], axis=0)
    b1a = jnp.concatenate([b1_s, b1_g], axis=0)
    W2a = jnp.concatenate([W2_s, W2_g], axis=0)
    b2a = jnp.concatenate([b2_s, b2_g], axis=0)

    yy = _mlp_table(x, W1a, b1a, W2a, b2a, bn=2000)
    gidx, sidx = _edge_indices(edge_index.astype(jnp.int32),
                               edge_type.astype(jnp.int32), n)
    gidx = gidx.reshape(2, n_sub, 2, 1, chunks // 2, csz)
    sidx = sidx.reshape(2, n_sub, 2, 1, chunks // 2, csz)
    kidx = jnp.concatenate([gidx, sidx], axis=3)
    n_pad = ((n + 8 * n_sub - 1) // (8 * n_sub)) * 8 * n_sub

    aggs = _make_sc_agg(n_pad, d, chunks, csz, n_sub)(yy, kidx)
    aggs = aggs[:, :n]

    out = _finish(x, aggs[0], aggs[1], Ws, bs, Wg, bg, pW, pb, oW, ob, bn=2000)
    return out.reshape(n)


# speedup vs baseline: 1.0719x; 1.0719x over previous
import functools

import jax
import jax.numpy as jnp
from jax import lax
from jax.experimental import pallas as pl
from jax.experimental.pallas import tpu as pltpu
from jax.experimental.pallas import tpu_sc as plsc

_CTR = (((1,), (1,)), ((), ()))


def _mlp_table_kernel(x_ref, w1_ref, b1_ref, w2_ref, b2_ref, out_ref):
    xb = x_ref[...]
    h = lax.dot_general(xb, w1_ref[0], _CTR, preferred_element_type=jnp.float32)
    h = jnp.maximum(h + b1_ref[0], 0.0)
    y = lax.dot_general(h, w2_ref[0], _CTR, preferred_element_type=jnp.float32)
    out_ref[...] = y + b2_ref[0]


def _mlp_table(x, W1a, b1a, W2a, b2a, bn):
    n, d = x.shape
    nk = W1a.shape[0]
    grid = (nk, n // bn)
    return pl.pallas_call(
        _mlp_table_kernel,
        grid=grid,
        in_specs=[
            pl.BlockSpec((bn, d), lambda k, i: (i, 0)),
            pl.BlockSpec((1, d, d), lambda k, i: (k, 0, 0)),
            pl.BlockSpec((1, 1, d), lambda k, i: (k, 0, 0)),
            pl.BlockSpec((1, d, d), lambda k, i: (k, 0, 0)),
            pl.BlockSpec((1, 1, d), lambda k, i: (k, 0, 0)),
        ],
        out_specs=pl.BlockSpec((bn, d), lambda k, i, _nb=n // bn: (k * _nb + i, 0)),
        out_shape=jax.ShapeDtypeStruct((nk * n, d), jnp.float32),
    )(x, W1a, b1a.reshape(nk, 1, d), W2a, b2a.reshape(nk, 1, d))


def _edge_idx_kernel(n_nodes, ei_ref, et_ref, gidx_ref, sidx_ref):
    src = ei_ref[0, :]
    dst = ei_ref[1, :]
    et = et_ref[...]
    gidx_ref[0, :] = et * n_nodes + src
    gidx_ref[1, :] = 2 * n_nodes + et * n_nodes + dst
    sidx_ref[0, :] = dst
    sidx_ref[1, :] = src


def _edge_indices(edge_index, edge_type, n_nodes):
    e = edge_type.shape[0]
    return pl.pallas_call(
        functools.partial(_edge_idx_kernel, n_nodes),
        out_shape=(
            jax.ShapeDtypeStruct((2, e), jnp.int32),
            jax.ShapeDtypeStruct((2, e), jnp.int32),
        ),
    )(edge_index, edge_type)


def _make_sc_agg(n_pad, d, chunks, csz, n_sub):
    rpt = n_pad // n_sub
    mesh = plsc.VectorSubcoreMesh(core_axis_name="c", subcore_axis_name="s")

    @functools.partial(
        pl.kernel,
        mesh=mesh,
        out_type=jax.ShapeDtypeStruct((2, n_pad, d), jnp.float32),
        scratch_types=[
            pltpu.VMEM((2, chunks // 2, csz), jnp.int32),
            pltpu.VMEM((csz, d), jnp.float32),
            pltpu.VMEM((csz, d), jnp.float32),
            pltpu.SemaphoreType.DMA,
            pltpu.SemaphoreType.DMA,
            pltpu.VMEM_SHARED((n_pad, d), jnp.float32),
        ],
    )
    def sc_agg(yy_hbm, kidx_hbm, out_hbm,
               idx_v, rows0, rows1, semg0, semg1, acc):
        c = lax.axis_index("c")
        s = lax.axis_index("s")
        zstrip = min(csz - csz % 8, rpt)
        zv = jnp.zeros((16,), jnp.float32)

        def zrow(r, carry):
            def zcol(q, carry2):
                rows0[r, pl.ds(q * 16, 16)] = zv
                return carry2
            return lax.fori_loop(0, d // 16, zcol, carry)

        lax.fori_loop(0, zstrip, zrow, 0)
        nfull = rpt // zstrip
        tail = rpt - nfull * zstrip

        def zdma(k, carry):
            pltpu.sync_copy(rows0.at[pl.ds(0, zstrip)],
                            acc.at[pl.ds(s * rpt + k * zstrip, zstrip)])
            return carry

        lax.fori_loop(0, nfull, zdma, 0)
        if tail:
            pltpu.sync_copy(rows0.at[pl.ds(0, tail)],
                            acc.at[pl.ds(s * rpt + nfull * zstrip, tail)])
        plsc.subcore_barrier()

        pchunks = chunks // 2

        def phase(p, carry):
            pltpu.sync_copy(kidx_hbm.at[c, s, p], idx_v)
            pltpu.async_copy(yy_hbm.at[idx_v.at[0, 0]], rows0, semg0)

            def body(j, carry2):
                i = 2 * j
                pltpu.make_async_copy(yy_hbm.at[idx_v.at[0, i]], rows0,
                                      semg0).wait()
                pltpu.async_copy(yy_hbm.at[idx_v.at[0, i + 1]], rows1, semg1)
                pltpu.sync_copy(rows0, acc.at[idx_v.at[1, i]], add=True)
                pltpu.make_async_copy(yy_hbm.at[idx_v.at[0, i + 1]], rows1,
                                      semg1).wait()

                @pl.when(i + 2 < pchunks)
                def _():
                    pltpu.async_copy(yy_hbm.at[idx_v.at[0, i + 2]], rows0,
                                     semg0)

                pltpu.sync_copy(rows1, acc.at[idx_v.at[1, i + 1]], add=True)
                return carry2

            lax.fori_loop(0, pchunks // 2, body, 0)
            return carry

        lax.fori_loop(0, 2, phase, 0)
        plsc.subcore_barrier()
        pltpu.sync_copy(acc.at[pl.ds(s * rpt, rpt)],
                        out_hbm.at[c, pl.ds(s * rpt, rpt)])

    return sc_agg


def _finish_kernel(x_ref, aggs_ref, aggg_ref, ws_ref, bs_ref, wg_ref, bg_ref,
                   pw_ref, pb_ref, ow_ref, ob_ref, out_ref):
    xb = x_ref[...]
    hu = jnp.tanh(lax.dot_general(xb, ws_ref[...], _CTR,
                                  preferred_element_type=jnp.float32)
                  + bs_ref[...] + aggs_ref[...])
    hv = jnp.tanh(lax.dot_general(xb, wg_ref[...], _CTR,
                                  preferred_element_type=jnp.float32)
                  + bg_ref[...] + aggg_ref[...])
    gate = jax.nn.sigmoid(
        jnp.dot(hu, pw_ref[...], preferred_element_type=jnp.float32)
        + pb_ref[0, 0])
    h = gate * hu + (1.0 - gate) * hv
    out_ref[...] = jax.nn.sigmoid(
        jnp.dot(h, ow_ref[...], preferred_element_type=jnp.float32)
        + ob_ref[0, 0])


def _finish(x, agg_s, agg_g, Ws, bs, Wg, bg, pW, pb, oW, ob, bn):
    n, d = x.shape
    full = pl.BlockSpec((d, d), lambda i: (0, 0))
    row = pl.BlockSpec((1, d), lambda i: (0, 0))
    col = pl.BlockSpec((d, 1), lambda i: (0, 0))
    scal = pl.BlockSpec((1, 1), lambda i: (0, 0))
    blk = pl.BlockSpec((bn, d), lambda i: (i, 0))
    return pl.pallas_call(
        _finish_kernel,
        grid=(n // bn,),
        in_specs=[blk, blk, blk, full, row, full, row, col, scal, col, scal],
        out_specs=pl.BlockSpec((bn, 1), lambda i: (i, 0)),
        out_shape=jax.ShapeDtypeStruct((n, 1), jnp.float32),
    )(x, agg_s, agg_g, Ws, bs.reshape(1, d), Wg, bg.reshape(1, d),
      pW.reshape(d, 1), pb.reshape(1, 1), oW.reshape(d, 1), ob.reshape(1, 1))


def kernel(x, edge_index, edge_type, W1_s, b1_s, W2_s, b2_s,
           W1_g, b1_g, W2_g, b2_g, Ws, bs, Wg, bg, pW, pb, oW, ob):
    n, d = x.shape
    e = edge_type.shape[0]
    n_sub = 16
    ept = e // n_sub
    csz = 125
    chunks = ept // csz

    W1a = jnp.concatenate([W1_s, W1_g], axis=0)
    b1a = jnp.concatenate([b1_s, b1_g], axis=0)
    W2a = jnp.concatenate([W2_s, W2_g], axis=0)
    b2a = jnp.concatenate([b2_s, b2_g], axis=0)

    yy = _mlp_table(x, W1a, b1a, W2a, b2a, bn=2000)
    gidx, sidx = _edge_indices(edge_index.astype(jnp.int32),
                               edge_type.astype(jnp.int32), n)
    gidx = gidx.reshape(2, n_sub, 2, 1, chunks // 2, csz)
    sidx = sidx.reshape(2, n_sub, 2, 1, chunks // 2, csz)
    kidx = jnp.concatenate([gidx, sidx], axis=3)
    n_pad = ((n + 8 * n_sub - 1) // (8 * n_sub)) * 8 * n_sub

    aggs = _make_sc_agg(n_pad, d, chunks, csz, n_sub)(yy, kidx)
    aggs = aggs[:, :n]

    out = _finish(x, aggs[0], aggs[1], Ws, bs, Wg, bg, pW, pb, oW, ob, bn=2000)
    return out.reshape(n)


# speedup vs baseline: 1.0864x; 1.0136x over previous
import functools

import jax
import jax.numpy as jnp
from jax import lax
from jax.experimental import pallas as pl
from jax.experimental.pallas import tpu as pltpu
from jax.experimental.pallas import tpu_sc as plsc

_CTR = (((1,), (1,)), ((), ()))


def _mlp_table_kernel(n_nodes, x_ref, w1_ref, b1_ref, w2_ref, b2_ref,
                      ei_ref, et_ref, out_ref, gidx_ref, sidx_ref):
    xb = x_ref[...]
    h = lax.dot_general(xb, w1_ref[0], _CTR, preferred_element_type=jnp.float32)
    h = jnp.maximum(h + b1_ref[0], 0.0)
    y = lax.dot_general(h, w2_ref[0], _CTR, preferred_element_type=jnp.float32)
    out_ref[...] = y + b2_ref[0]

    @pl.when(jnp.logical_and(pl.program_id(0) == 0, pl.program_id(1) == 0))
    def _():
        src = ei_ref[0, :]
        dst = ei_ref[1, :]
        et = et_ref[0, :]
        gidx_ref[0, :] = et * n_nodes + src
        gidx_ref[1, :] = 2 * n_nodes + et * n_nodes + dst
        sidx_ref[0, :] = dst
        sidx_ref[1, :] = src


def _mlp_table(x, W1a, b1a, W2a, b2a, edge_index, edge_type, bn):
    n, d = x.shape
    e = edge_type.shape[0]
    nk = W1a.shape[0]
    grid = (nk, n // bn)
    full2 = pl.BlockSpec((2, e), lambda k, i: (0, 0))
    return pl.pallas_call(
        functools.partial(_mlp_table_kernel, n),
        grid=grid,
        in_specs=[
            pl.BlockSpec((bn, d), lambda k, i: (i, 0)),
            pl.BlockSpec((1, d, d), lambda k, i: (k, 0, 0)),
            pl.BlockSpec((1, 1, d), lambda k, i: (k, 0, 0)),
            pl.BlockSpec((1, d, d), lambda k, i: (k, 0, 0)),
            pl.BlockSpec((1, 1, d), lambda k, i: (k, 0, 0)),
            full2,
            pl.BlockSpec((1, e), lambda k, i: (0, 0)),
        ],
        out_specs=[
            pl.BlockSpec((bn, d), lambda k, i, _nb=n // bn: (k * _nb + i, 0)),
            full2,
            full2,
        ],
        out_shape=[
            jax.ShapeDtypeStruct((nk * n, d), jnp.float32),
            jax.ShapeDtypeStruct((2, e), jnp.int32),
            jax.ShapeDtypeStruct((2, e), jnp.int32),
        ],
    )(x, W1a, b1a.reshape(nk, 1, d), W2a, b2a.reshape(nk, 1, d),
      edge_index, edge_type.reshape(1, e))


def _make_sc_agg(n_pad, d, chunks, csz, n_sub):
    rpt = n_pad // n_sub
    mesh = plsc.VectorSubcoreMesh(core_axis_name="c", subcore_axis_name="s")

    @functools.partial(
        pl.kernel,
        mesh=mesh,
        out_type=jax.ShapeDtypeStruct((2, n_pad, d), jnp.float32),
        scratch_types=[
            pltpu.VMEM((2, chunks // 2, csz), jnp.int32),
            pltpu.VMEM((csz, d), jnp.float32),
            pltpu.VMEM((csz, d), jnp.float32),
            pltpu.SemaphoreType.DMA,
            pltpu.SemaphoreType.DMA,
            pltpu.VMEM_SHARED((n_pad, d), jnp.float32),
        ],
    )
    def sc_agg(yy_hbm, kidx_hbm, out_hbm,
               idx_v, rows0, rows1, semg0, semg1, acc):
        c = lax.axis_index("c")
        s = lax.axis_index("s")
        zstrip = min(csz - csz % 8, rpt)
        zv = jnp.zeros((16,), jnp.float32)

        def zrow(r, carry):
            def zcol(q, carry2):
                rows0[r, pl.ds(q * 16, 16)] = zv
                return carry2
            return lax.fori_loop(0, d // 16, zcol, carry)

        lax.fori_loop(0, zstrip, zrow, 0)
        nfull = rpt // zstrip
        tail = rpt - nfull * zstrip

        def zdma(k, carry):
            pltpu.sync_copy(rows0.at[pl.ds(0, zstrip)],
                            acc.at[pl.ds(s * rpt + k * zstrip, zstrip)])
            return carry

        lax.fori_loop(0, nfull, zdma, 0)
        if tail:
            pltpu.sync_copy(rows0.at[pl.ds(0, tail)],
                            acc.at[pl.ds(s * rpt + nfull * zstrip, tail)])
        plsc.subcore_barrier()

        pchunks = chunks // 2

        def phase(p, carry):
            pltpu.sync_copy(kidx_hbm.at[c, s, p], idx_v)
            pltpu.async_copy(yy_hbm.at[idx_v.at[0, 0]], rows0, semg0)

            def body(j, carry2):
                i = 2 * j
                pltpu.make_async_copy(yy_hbm.at[idx_v.at[0, i]], rows0,
                                      semg0).wait()
                pltpu.async_copy(yy_hbm.at[idx_v.at[0, i + 1]], rows1, semg1)
                pltpu.sync_copy(rows0, acc.at[idx_v.at[1, i]], add=True)
                pltpu.make_async_copy(yy_hbm.at[idx_v.at[0, i + 1]], rows1,
                                      semg1).wait()

                @pl.when(i + 2 < pchunks)
                def _():
                    pltpu.async_copy(yy_hbm.at[idx_v.at[0, i + 2]], rows0,
                                     semg0)

                pltpu.sync_copy(rows1, acc.at[idx_v.at[1, i + 1]], add=True)
                return carry2

            lax.fori_loop(0, pchunks // 2, body, 0)
            return carry

        lax.fori_loop(0, 2, phase, 0)
        plsc.subcore_barrier()
        pltpu.sync_copy(acc.at[pl.ds(s * rpt, rpt)],
                        out_hbm.at[c, pl.ds(s * rpt, rpt)])

    return sc_agg


def _finish_kernel(x_ref, aggs_ref, aggg_ref, ws_ref, bs_ref, wg_ref, bg_ref,
                   pw_ref, pb_ref, ow_ref, ob_ref, out_ref):
    xb = x_ref[...]
    hu = jnp.tanh(lax.dot_general(xb, ws_ref[...], _CTR,
                                  preferred_element_type=jnp.float32)
                  + bs_ref[...] + aggs_ref[0])
    hv = jnp.tanh(lax.dot_general(xb, wg_ref[...], _CTR,
                                  preferred_element_type=jnp.float32)
                  + bg_ref[...] + aggg_ref[0])
    gate = jax.nn.sigmoid(
        jnp.dot(hu, pw_ref[...], preferred_element_type=jnp.float32)
        + pb_ref[0, 0])
    h = gate * hu + (1.0 - gate) * hv
    out_ref[...] = jax.nn.sigmoid(
        jnp.dot(h, ow_ref[...], preferred_element_type=jnp.float32)
        + ob_ref[0, 0])


def _finish(x, aggs, Ws, bs, Wg, bg, pW, pb, oW, ob, bn):
    n, d = x.shape
    full = pl.BlockSpec((d, d), lambda i: (0, 0))
    row = pl.BlockSpec((1, d), lambda i: (0, 0))
    col = pl.BlockSpec((d, 1), lambda i: (0, 0))
    scal = pl.BlockSpec((1, 1), lambda i: (0, 0))
    blk = pl.BlockSpec((bn, d), lambda i: (i, 0))
    agg0 = pl.BlockSpec((1, bn, d), lambda i: (0, i, 0))
    agg1 = pl.BlockSpec((1, bn, d), lambda i: (1, i, 0))
    return pl.pallas_call(
        _finish_kernel,
        grid=(n // bn,),
        in_specs=[blk, agg0, agg1, full, row, full, row, col, scal, col,
                  scal],
        out_specs=pl.BlockSpec((bn, 1), lambda i: (i, 0)),
        out_shape=jax.ShapeDtypeStruct((n, 1), jnp.float32),
    )(x, aggs, aggs, Ws, bs.reshape(1, d), Wg, bg.reshape(1, d),
      pW.reshape(d, 1), pb.reshape(1, 1), oW.reshape(d, 1), ob.reshape(1, 1))


def kernel(x, edge_index, edge_type, W1_s, b1_s, W2_s, b2_s,
           W1_g, b1_g, W2_g, b2_g, Ws, bs, Wg, bg, pW, pb, oW, ob):
    n, d = x.shape
    e = edge_type.shape[0]
    n_sub = 16
    ept = e // n_sub
    csz = 125
    chunks = ept // csz

    W1a = jnp.concatenate([W1_s, W1_g], axis=0)
    b1a = jnp.concatenate([b1_s, b1_g], axis=0)
    W2a = jnp.concatenate([W2_s, W2_g], axis=0)
    b2a = jnp.concatenate([b2_s, b2_g], axis=0)

    yy, gidx, sidx = _mlp_table(x, W1a, b1a, W2a, b2a,
                                edge_index.astype(jnp.int32),
                                edge_type.astype(jnp.int32), bn=2000)
    gidx = gidx.reshape(2, n_sub, 2, 1, chunks // 2, csz)
    sidx = sidx.reshape(2, n_sub, 2, 1, chunks // 2, csz)
    kidx = jnp.concatenate([gidx, sidx], axis=3)
    n_pad = ((n + 8 * n_sub - 1) // (8 * n_sub)) * 8 * n_sub

    aggs = _make_sc_agg(n_pad, d, chunks, csz, n_sub)(yy, kidx)

    out = _finish(x, aggs, Ws, bs, Wg, bg, pW, pb, oW, ob, bn=2000)
    return out.reshape(n)


# speedup vs baseline: 1.1549x; 1.0630x over previous
import functools

import jax
import jax.numpy as jnp
from jax import lax
from jax.experimental import pallas as pl
from jax.experimental.pallas import tpu as pltpu
from jax.experimental.pallas import tpu_sc as plsc

_CTR = (((1,), (1,)), ((), ()))


def _mlp_table_kernel(n_nodes, x_ref, w1_ref, b1_ref, w2_ref, b2_ref,
                      ei_ref, et_ref, out_ref, gidx_ref, sidx_ref):
    xb = x_ref[...]
    h = lax.dot_general(xb, w1_ref[0], _CTR, preferred_element_type=jnp.float32)
    h = jnp.maximum(h + b1_ref[0], 0.0)
    y = lax.dot_general(h, w2_ref[0], _CTR, preferred_element_type=jnp.float32)
    out_ref[...] = y + b2_ref[0]

    @pl.when(jnp.logical_and(pl.program_id(0) == 0, pl.program_id(1) == 0))
    def _():
        src = ei_ref[0, :]
        dst = ei_ref[1, :]
        et = et_ref[0, :]
        gidx_ref[0, :] = et * n_nodes + src
        gidx_ref[1, :] = 2 * n_nodes + et * n_nodes + dst
        sidx_ref[0, :] = dst
        sidx_ref[1, :] = src


def _mlp_table(x, W1a, b1a, W2a, b2a, edge_index, edge_type, bn):
    n, d = x.shape
    e = edge_type.shape[0]
    nk = W1a.shape[0]
    grid = (nk, n // bn)
    full2 = pl.BlockSpec((2, e), lambda k, i: (0, 0))
    return pl.pallas_call(
        functools.partial(_mlp_table_kernel, n),
        grid=grid,
        in_specs=[
            pl.BlockSpec((bn, d), lambda k, i: (i, 0)),
            pl.BlockSpec((1, d, d), lambda k, i: (k, 0, 0)),
            pl.BlockSpec((1, 1, d), lambda k, i: (k, 0, 0)),
            pl.BlockSpec((1, d, d), lambda k, i: (k, 0, 0)),
            pl.BlockSpec((1, 1, d), lambda k, i: (k, 0, 0)),
            full2,
            pl.BlockSpec((1, e), lambda k, i: (0, 0)),
        ],
        out_specs=[
            pl.BlockSpec((bn, d), lambda k, i, _nb=n // bn: (k * _nb + i, 0)),
            full2,
            full2,
        ],
        out_shape=[
            jax.ShapeDtypeStruct((nk * n, d), jnp.float32),
            jax.ShapeDtypeStruct((2, e), jnp.int32),
            jax.ShapeDtypeStruct((2, e), jnp.int32),
        ],
    )(x, W1a, b1a.reshape(nk, 1, d), W2a, b2a.reshape(nk, 1, d),
      edge_index, edge_type.reshape(1, e))


def _make_sc_agg(n_pad, d, chunks, csz, n_sub):
    rpt = n_pad // n_sub
    mesh = plsc.VectorSubcoreMesh(core_axis_name="c", subcore_axis_name="s")

    @functools.partial(
        pl.kernel,
        mesh=mesh,
        out_type=jax.ShapeDtypeStruct((2, n_pad, d), jnp.float32),
        scratch_types=[
            pltpu.VMEM((2, chunks // 2, csz), jnp.int32),
            pltpu.VMEM((csz, d), jnp.float32),
            pltpu.VMEM((csz, d), jnp.float32),
            pltpu.SemaphoreType.DMA,
            pltpu.SemaphoreType.DMA,
            pltpu.VMEM_SHARED((n_pad, d), jnp.float32),
        ],
    )
    def sc_agg(yy_hbm, kidx_hbm, out_hbm,
               idx_v, rows0, rows1, semg0, semg1, acc):
        c = lax.axis_index("c")
        s = lax.axis_index("s")
        zstrip = min(csz - csz % 8, rpt)
        zv = jnp.zeros((16,), jnp.float32)

        def zrow(r, carry):
            def zcol(q, carry2):
                rows0[r, pl.ds(q * 16, 16)] = zv
                return carry2
            return lax.fori_loop(0, d // 16, zcol, carry)

        lax.fori_loop(0, zstrip, zrow, 0)
        nfull = rpt // zstrip
        tail = rpt - nfull * zstrip

        def zdma(k, carry):
            pltpu.sync_copy(rows0.at[pl.ds(0, zstrip)],
                            acc.at[pl.ds(s * rpt + k * zstrip, zstrip)])
            return carry

        lax.fori_loop(0, nfull, zdma, 0)
        if tail:
            pltpu.sync_copy(rows0.at[pl.ds(0, tail)],
                            acc.at[pl.ds(s * rpt + nfull * zstrip, tail)])
        plsc.subcore_barrier()

        pchunks = chunks // 2

        def phase(p, carry):
            pltpu.sync_copy(kidx_hbm.at[c, s, p], idx_v)
            pltpu.async_copy(yy_hbm.at[idx_v.at[0, 0]], rows0, semg0)

            def body(j, carry2):
                i = 2 * j
                pltpu.make_async_copy(yy_hbm.at[idx_v.at[0, i]], rows0,
                                      semg0).wait()
                pltpu.async_copy(yy_hbm.at[idx_v.at[0, i + 1]], rows1, semg1)
                pltpu.sync_copy(rows0, acc.at[idx_v.at[1, i]], add=True)
                pltpu.make_async_copy(yy_hbm.at[idx_v.at[0, i + 1]], rows1,
                                      semg1).wait()

                @pl.when(i + 2 < pchunks)
                def _():
                    pltpu.async_copy(yy_hbm.at[idx_v.at[0, i + 2]], rows0,
                                     semg0)

                pltpu.sync_copy(rows1, acc.at[idx_v.at[1, i + 1]], add=True)
                return carry2

            lax.fori_loop(0, pchunks // 2, body, 0)
            return carry

        lax.fori_loop(0, 2, phase, 0)
        plsc.subcore_barrier()
        pltpu.sync_copy(acc.at[pl.ds(s * rpt, rpt)],
                        out_hbm.at[c, pl.ds(s * rpt, rpt)])

    return sc_agg


def _finish_kernel(x_ref, aggs_ref, aggg_ref, ws_ref, bs_ref, wg_ref, bg_ref,
                   pw_ref, pb_ref, ow_ref, ob_ref, out_ref):
    xb = x_ref[...]
    hu = jnp.tanh(lax.dot_general(xb, ws_ref[...], _CTR,
                                  preferred_element_type=jnp.float32)
                  + bs_ref[...] + aggs_ref[0])
    hv = jnp.tanh(lax.dot_general(xb, wg_ref[...], _CTR,
                                  preferred_element_type=jnp.float32)
                  + bg_ref[...] + aggg_ref[0])
    gate = jax.nn.sigmoid(
        jnp.dot(hu, pw_ref[...], preferred_element_type=jnp.float32)
        + pb_ref[0, 0])
    h = gate * hu + (1.0 - gate) * hv
    out_ref[...] = jax.nn.sigmoid(
        jnp.dot(h, ow_ref[...], preferred_element_type=jnp.float32)
        + ob_ref[0, 0])


def _finish(x, aggs, Ws, bs, Wg, bg, pW, pb, oW, ob, bn):
    n, d = x.shape
    full = pl.BlockSpec((d, d), lambda i: (0, 0))
    row = pl.BlockSpec((1, d), lambda i: (0, 0))
    col = pl.BlockSpec((d, 1), lambda i: (0, 0))
    scal = pl.BlockSpec((1, 1), lambda i: (0, 0))
    blk = pl.BlockSpec((bn, d), lambda i: (i, 0))
    agg0 = pl.BlockSpec((1, bn, d), lambda i: (0, i, 0))
    agg1 = pl.BlockSpec((1, bn, d), lambda i: (1, i, 0))
    return pl.pallas_call(
        _finish_kernel,
        grid=(n // bn,),
        in_specs=[blk, agg0, agg1, full, row, full, row, col, scal, col,
                  scal],
        out_specs=pl.BlockSpec((bn, 1), lambda i: (i, 0)),
        out_shape=jax.ShapeDtypeStruct((n, 1), jnp.float32),
    )(x, aggs, aggs, Ws, bs.reshape(1, d), Wg, bg.reshape(1, d),
      pW.reshape(d, 1), pb.reshape(1, 1), oW.reshape(d, 1), ob.reshape(1, 1))


def kernel(x, edge_index, edge_type, W1_s, b1_s, W2_s, b2_s,
           W1_g, b1_g, W2_g, b2_g, Ws, bs, Wg, bg, pW, pb, oW, ob):
    n, d = x.shape
    e = edge_type.shape[0]
    n_sub = 16
    ept = e // n_sub
    csz = 125
    chunks = ept // csz

    W1a = jnp.concatenate([W1_s, W1_g], axis=0)
    b1a = jnp.concatenate([b1_s, b1_g], axis=0)
    W2a = jnp.concatenate([W2_s, W2_g], axis=0)
    b2a = jnp.concatenate([b2_s, b2_g], axis=0)

    yy, gidx, sidx = _mlp_table(x, W1a, b1a, W2a, b2a,
                                edge_index.astype(jnp.int32),
                                edge_type.astype(jnp.int32), bn=10000)
    gidx = gidx.reshape(2, n_sub, 2, 1, chunks // 2, csz)
    sidx = sidx.reshape(2, n_sub, 2, 1, chunks // 2, csz)
    kidx = jnp.concatenate([gidx, sidx], axis=3)
    n_pad = ((n + 8 * n_sub - 1) // (8 * n_sub)) * 8 * n_sub

    aggs = _make_sc_agg(n_pad, d, chunks, csz, n_sub)(yy, kidx)

    out = _finish(x, aggs, Ws, bs, Wg, bg, pW, pb, oW, ob, bn=10000)
    return out.reshape(n)
